# Initial kernel scaffold; baseline (speedup 1.0000x reference)
import jax
import jax.numpy as jnp
from jax.experimental import pallas as pl


def kernel(states, W, b):
    raise NotImplementedError("write your pallas kernel here")


# speedup vs baseline: 3.2410x; 3.2410x over previous
import functools

import jax
import jax.numpy as jnp
from jax import lax
from jax.experimental import pallas as pl
from jax.experimental.pallas import tpu as pltpu
from jax.experimental.pallas import tpu_sc as plsc

_STATE_FEATURES = 26
_BUCKETS = 1000
_N = 4096 * _STATE_FEATURES
_NC, _NS, _L = 2, 16, 16
_NW = _NC * _NS
_PER_W = _N // _NW
_ITERS = _PER_W // _L

_mesh = plsc.VectorSubcoreMesh(core_axis_name="c", subcore_axis_name="s")


@functools.partial(
    pl.kernel,
    mesh=_mesh,
    out_type=jax.ShapeDtypeStruct((_N,), jnp.float32),
    scratch_types=[
        pltpu.VMEM((_PER_W,), jnp.float32),
        pltpu.VMEM((_BUCKETS,), jnp.float32),
        pltpu.VMEM((_L,), jnp.float32),
        pltpu.VMEM((_PER_W,), jnp.float32),
    ],
    compiler_params=pltpu.CompilerParams(needs_layout_passes=False),
)
def _lookup(states_hbm, table_hbm, bias_hbm, out_hbm, x_v, tab_v, b_v, o_v):
    wid = lax.axis_index("s") * _NC + lax.axis_index("c")
    base = wid * _PER_W
    pltpu.sync_copy(states_hbm.at[pl.ds(base, _PER_W)], x_v)
    pltpu.sync_copy(table_hbm, tab_v)
    pltpu.sync_copy(bias_hbm, b_v)
    bias = b_v[...]

    def body(i, carry):
        x = x_v[pl.ds(i * _L, _L)]
        idx = (x * float(_BUCKETS - 1)).astype(jnp.int32)
        vals = plsc.load_gather(tab_v, [idx])
        o_v[pl.ds(i * _L, _L)] = vals + bias
        return carry

    lax.fori_loop(0, _ITERS, body, 0)
    pltpu.sync_copy(o_v, out_hbm.at[pl.ds(base, _PER_W)])


def kernel(states, W, b):
    x = states.reshape(-1)
    table = W.reshape(-1)
    bias = jnp.broadcast_to(b.reshape(1), (_L,))
    out = _lookup(x, table, bias)
    return out.reshape(-1, _STATE_FEATURES)


# speedup vs baseline: 3.4227x; 1.0561x over previous
import functools

import jax
import jax.numpy as jnp
from jax import lax
from jax.experimental import pallas as pl
from jax.experimental.pallas import tpu as pltpu
from jax.experimental.pallas import tpu_sc as plsc

_STATE_FEATURES = 26
_BUCKETS = 1000
_N = 4096 * _STATE_FEATURES
_NC, _NS, _L = 2, 16, 16
_NW = _NC * _NS
_PER_W = _N // _NW
_ITERS = _PER_W // _L

_mesh = plsc.VectorSubcoreMesh(core_axis_name="c", subcore_axis_name="s")


@functools.partial(
    pl.kernel,
    mesh=_mesh,
    out_type=jax.ShapeDtypeStruct((_N,), jnp.float32),
    scratch_types=[
        pltpu.VMEM((_PER_W,), jnp.float32),
        pltpu.VMEM((_BUCKETS,), jnp.float32),
        pltpu.VMEM((_L,), jnp.float32),
        pltpu.VMEM((_PER_W,), jnp.float32),
        pltpu.SemaphoreType.DMA,
    ],
    compiler_params=pltpu.CompilerParams(needs_layout_passes=False),
)
def _lookup(states_hbm, table_hbm, bias_hbm, out_hbm, x_v, tab_v, b_v, o_v, sem):
    wid = lax.axis_index("s") * _NC + lax.axis_index("c")
    base = wid * _PER_W
    c1 = pltpu.make_async_copy(states_hbm.at[pl.ds(base, _PER_W)], x_v, sem)
    c2 = pltpu.make_async_copy(table_hbm, tab_v, sem)
    c3 = pltpu.make_async_copy(bias_hbm, b_v, sem)
    c1.start()
    c2.start()
    c3.start()
    c1.wait()
    c2.wait()
    c3.wait()
    bias = b_v[...]

    @plsc.parallel_loop(0, _PER_W, step=_L, unroll=8)
    def _(off):
        x = x_v[pl.ds(off, _L)]
        idx = (x * float(_BUCKETS - 1)).astype(jnp.int32)
        o_v[pl.ds(off, _L)] = plsc.load_gather(tab_v, [idx]) + bias

    pltpu.sync_copy(o_v, out_hbm.at[pl.ds(base, _PER_W)])


def kernel(states, W, b):
    x = states.reshape(-1)
    table = W.reshape(-1)
    bias = jnp.broadcast_to(b.reshape(1), (_L,))
    out = _lookup(x, table, bias)
    return out.reshape(-1, _STATE_FEATURES)


# speedup vs baseline: 3.6685x; 1.0718x over previous
import functools

import jax
import jax.numpy as jnp
from jax import lax
from jax.experimental import pallas as pl
from jax.experimental.pallas import tpu as pltpu
from jax.experimental.pallas import tpu_sc as plsc

_ROWS = 4096
_COLS = 26
_BUCKETS = 1000
_NC, _NS, _L = 2, 16, 16
_NW = _NC * _NS
_RPW = _ROWS // _NW

_mesh = plsc.VectorSubcoreMesh(core_axis_name="c", subcore_axis_name="s")


@functools.partial(
    pl.kernel,
    mesh=_mesh,
    out_type=jax.ShapeDtypeStruct((_ROWS, _COLS), jnp.float32),
    scratch_types=[
        pltpu.VMEM((_RPW, _COLS), jnp.float32),
        pltpu.VMEM((_BUCKETS,), jnp.float32),
        pltpu.VMEM((_L,), jnp.float32),
        pltpu.VMEM((_RPW, _COLS), jnp.float32),
        pltpu.SemaphoreType.DMA,
    ],
    compiler_params=pltpu.CompilerParams(needs_layout_passes=False),
)
def _lookup(states_hbm, table_hbm, bias_hbm, out_hbm, x_v, tab_v, b_v, o_v, sem):
    wid = lax.axis_index("s") * _NC + lax.axis_index("c")
    base = wid * _RPW
    c1 = pltpu.make_async_copy(states_hbm.at[pl.ds(base, _RPW)], x_v, sem)
    c2 = pltpu.make_async_copy(table_hbm, tab_v, sem)
    c3 = pltpu.make_async_copy(bias_hbm, b_v, sem)
    c1.start()
    c2.start()
    c3.start()
    c1.wait()
    c2.wait()
    c3.wait()
    bias = b_v[...]

    @plsc.parallel_loop(0, _RPW, step=1, unroll=4)
    def _(r):
        xa = x_v[r, pl.ds(0, _L)]
        xb = x_v[r, pl.ds(_COLS - _L, _L)]
        ia = (xa * float(_BUCKETS - 1)).astype(jnp.int32)
        ib = (xb * float(_BUCKETS - 1)).astype(jnp.int32)
        o_v[r, pl.ds(0, _L)] = plsc.load_gather(tab_v, [ia]) + bias
        o_v[r, pl.ds(_COLS - _L, _L)] = plsc.load_gather(tab_v, [ib]) + bias

    pltpu.sync_copy(o_v, out_hbm.at[pl.ds(base, _RPW)])


def kernel(states, W, b):
    table = W.reshape(-1)
    bias = jnp.broadcast_to(b.reshape(1), (_L,))
    return _lookup(states, table, bias)


# speedup vs baseline: 3.6758x; 1.0020x over previous
import functools

import jax
import jax.numpy as jnp
from jax import lax
from jax.experimental import pallas as pl
from jax.experimental.pallas import tpu as pltpu
from jax.experimental.pallas import tpu_sc as plsc

_ROWS = 4096
_COLS = 26
_BUCKETS = 1000
_NC, _NS, _L = 2, 16, 16
_NW = _NC * _NS
_RPW = _ROWS // _NW

_mesh = plsc.VectorSubcoreMesh(core_axis_name="c", subcore_axis_name="s")


@functools.partial(
    pl.kernel,
    mesh=_mesh,
    out_type=jax.ShapeDtypeStruct((_ROWS, _COLS), jnp.float32),
    scratch_types=[
        pltpu.VMEM((_RPW, _COLS), jnp.float32),
        pltpu.VMEM((_BUCKETS,), jnp.float32),
        pltpu.VMEM((_L,), jnp.float32),
        pltpu.VMEM((_RPW, _COLS), jnp.float32),
        pltpu.SemaphoreType.DMA,
    ],
    compiler_params=pltpu.CompilerParams(
        needs_layout_passes=False, use_tc_tiling_on_sc=True
    ),
)
def _lookup(states_hbm, table_hbm, bias_hbm, out_hbm, x_v, tab_v, b_v, o_v, sem):
    wid = lax.axis_index("s") * _NC + lax.axis_index("c")
    base = wid * _RPW
    c1 = pltpu.make_async_copy(states_hbm.at[pl.ds(base, _RPW)], x_v, sem)
    c2 = pltpu.make_async_copy(table_hbm, tab_v, sem)
    c3 = pltpu.make_async_copy(bias_hbm, b_v, sem)
    c1.start()
    c2.start()
    c3.start()
    c1.wait()
    c2.wait()
    c3.wait()
    bias = b_v[...]

    @plsc.parallel_loop(0, _RPW, step=1, unroll=4)
    def _(r):
        xa = x_v[r, pl.ds(0, _L)]
        xb = x_v[r, pl.ds(_COLS - _L, _L)]
        ia = (xa * float(_BUCKETS - 1)).astype(jnp.int32)
        ib = (xb * float(_BUCKETS - 1)).astype(jnp.int32)
        o_v[r, pl.ds(0, _L)] = plsc.load_gather(tab_v, [ia]) + bias
        o_v[r, pl.ds(_COLS - _L, _L)] = plsc.load_gather(tab_v, [ib]) + bias

    pltpu.sync_copy(o_v, out_hbm.at[pl.ds(base, _RPW)])


def kernel(states, W, b):
    table = W.reshape(-1)
    bias = jnp.broadcast_to(b.reshape(1), (_L,))
    return _lookup(states, table, bias)


# speedup vs baseline: 3.7819x; 1.0289x over previous
import functools

import jax
import jax.numpy as jnp
from jax import lax
from jax.experimental import pallas as pl
from jax.experimental.pallas import tpu as pltpu
from jax.experimental.pallas import tpu_sc as plsc

_ROWS = 4096
_COLS = 26
_BUCKETS = 1000
_NC, _NS, _L = 2, 16, 16
_NW = _NC * _NS
_RPW = _ROWS // _NW

_mesh = plsc.VectorSubcoreMesh(core_axis_name="c", subcore_axis_name="s")


@functools.partial(
    pl.kernel,
    mesh=_mesh,
    out_type=jax.ShapeDtypeStruct((_ROWS, _COLS), jnp.float32),
    scratch_types=[
        pltpu.VMEM((_RPW, _COLS), jnp.float32),
        pltpu.VMEM((_BUCKETS,), jnp.float32),
        pltpu.VMEM((_L,), jnp.float32),
        pltpu.VMEM((_RPW, _COLS), jnp.float32),
        pltpu.SemaphoreType.DMA,
    ],
    compiler_params=pltpu.CompilerParams(
        needs_layout_passes=False, use_tc_tiling_on_sc=False
    ),
)
def _lookup(states_hbm, table_hbm, bias_hbm, out_hbm, x_v, tab_v, b_v, o_v, sem):
    wid = lax.axis_index("s") * _NC + lax.axis_index("c")
    base = wid * _RPW
    c1 = pltpu.make_async_copy(states_hbm.at[pl.ds(base, _RPW)], x_v, sem)
    c2 = pltpu.make_async_copy(table_hbm, tab_v, sem)
    c3 = pltpu.make_async_copy(bias_hbm, b_v, sem)
    c1.start()
    c2.start()
    c3.start()
    c1.wait()
    c2.wait()
    c3.wait()
    bias = b_v[...]

    @plsc.parallel_loop(0, _RPW, step=1, unroll=4)
    def _(r):
        xa = x_v[r, pl.ds(0, _L)]
        xb = x_v[r, pl.ds(_COLS - _L, _L)]
        ia = (xa * float(_BUCKETS - 1)).astype(jnp.int32)
        ib = (xb * float(_BUCKETS - 1)).astype(jnp.int32)
        o_v[r, pl.ds(0, _L)] = plsc.load_gather(tab_v, [ia]) + bias
        o_v[r, pl.ds(_COLS - _L, _L)] = plsc.load_gather(tab_v, [ib]) + bias

    pltpu.sync_copy(o_v, out_hbm.at[pl.ds(base, _RPW)])


def kernel(states, W, b):
    table = W.reshape(-1)
    bias = jnp.broadcast_to(b.reshape(1), (_L,))
    return _lookup(states, table, bias)
